# baseline (speedup 1.0000x reference)
import functools

import jax
import jax.numpy as jnp
from jax.experimental import pallas as pl
from jax.experimental.pallas import tpu as pltpu

B = 512
D = 256
H = 512
S = 64
UNROLL = 16
NBLK = S // UNROLL
BSPLIT = 2
BB = B // BSPLIT


def _cell_kernel(
    x_ref,
    slots_ref,
    ptr_ref,
    wv_ref,
    bv_ref,
    wdet_ref,
    bdet_ref,
    pos_ref,
    wcat_ref,
    bias_ref,
    h_out_ref,
    ns_out_ref,
    np_out_ref,
    v_ref,
    m_ref,
    hb_ref,
    c_ref,
    pg_ref,
    xt_ref,
    gx_ref,
):
    t = pl.program_id(1)

    @pl.when(t == 0)
    def _prologue():
        x = x_ref[...]
        logit = jnp.sum(x * wdet_ref[...], axis=1, keepdims=True) + bdet_ref[...]
        ev = (jax.nn.sigmoid(logit) > 0.85)
        m_ref[...] = ev.astype(jnp.float32)
        v_ref[...] = jnp.dot(x, wv_ref[...], preferred_element_type=jnp.float32) + bv_ref[...]
        ptr = ptr_ref[...]
        np_out_ref[...] = jax.lax.rem(ptr + ev.astype(jnp.int32), jnp.int32(S))
        hb_ref[...] = jnp.zeros_like(hb_ref)
        c_ref[...] = jnp.zeros_like(c_ref)
        pg_ref[...] = bias_ref[...] + jnp.dot(
            pos_ref[...].astype(jnp.bfloat16), wcat_ref[0:D, :],
            preferred_element_type=jnp.float32)

    kidx = jax.lax.broadcasted_iota(jnp.int32, (BB, UNROLL, 1), 1) + t * UNROLL
    cond = jnp.logical_and(ptr_ref[...][:, :, None] == kidx,
                           m_ref[...][:, :, None] > 0.5)
    ns_block = jnp.where(cond, v_ref[...][:, None, :], slots_ref[...])
    ns_out_ref[...] = ns_block
    xt_ref[...] = jnp.swapaxes(ns_block.astype(jnp.bfloat16), 0, 1)
    gx_ref[...] = jnp.dot(
        xt_ref[...].reshape(UNROLL * BB, D), wcat_ref[0:D, :],
        preferred_element_type=jnp.float32,
    ).astype(jnp.bfloat16).reshape(UNROLL, BB, 4 * H)

    for k in range(UNROLL):
        s = t * UNROLL + k
        gates = (
            jnp.dot(hb_ref[...], wcat_ref[D:D + H, :], preferred_element_type=jnp.float32)
            + gx_ref[k].astype(jnp.float32)
            + pg_ref[pl.ds(s, 1), :]
        )
        tg = jnp.tanh(gates)
        i = 0.5 * tg[:, 0 * H:1 * H] + 0.5
        f = 0.5 * tg[:, 1 * H:2 * H] + 0.5
        g = tg[:, 2 * H:3 * H]
        o = 0.5 * tg[:, 3 * H:4 * H] + 0.5
        c = f * c_ref[...] + i * g
        c_ref[...] = c
        h_new = o * jnp.tanh(c)
        hb_ref[...] = h_new.astype(jnp.bfloat16)
        if k == UNROLL - 1:
            @pl.when(t == NBLK - 1)
            def _epilogue():
                h_out_ref[...] = h_new


@functools.partial(jax.jit, static_argnames=("interpret",))
def _run(x_t, slots, ptr2, wv_t, b_v, W_det, bdet, pos_emb, wcat, bias, interpret=False):
    shared = lambda shape: pl.BlockSpec(shape, lambda i, t: (0,) * len(shape))
    bhalf = lambda shape: pl.BlockSpec(shape, lambda i, t: (i,) + (0,) * (len(shape) - 1))
    out = pl.pallas_call(
        _cell_kernel,
        grid=(BSPLIT, NBLK),
        in_specs=[
            bhalf((BB, D)),
            pl.BlockSpec((BB, UNROLL, D), lambda i, t: (i, t, 0)),
            bhalf((BB, 1)),
            shared((D, D)),
            shared((1, D)),
            shared((1, D)),
            shared((1, 1)),
            shared((S, D)),
            shared((D + H, 4 * H)),
            shared((1, 4 * H)),
        ],
        out_specs=[
            bhalf((BB, H)),
            pl.BlockSpec((BB, UNROLL, D), lambda i, t: (i, t, 0)),
            bhalf((BB, 1)),
        ],
        out_shape=[
            jax.ShapeDtypeStruct((B, H), jnp.float32),
            jax.ShapeDtypeStruct((B, S, D), jnp.float32),
            jax.ShapeDtypeStruct((B, 1), jnp.int32),
        ],
        scratch_shapes=[
            pltpu.VMEM((BB, D), jnp.float32),
            pltpu.VMEM((BB, 1), jnp.float32),
            pltpu.VMEM((BB, H), jnp.bfloat16),
            pltpu.VMEM((BB, H), jnp.float32),
            pltpu.VMEM((S, 4 * H), jnp.float32),
            pltpu.VMEM((UNROLL, BB, D), jnp.bfloat16),
            pltpu.VMEM((UNROLL, BB, 4 * H), jnp.bfloat16),
        ],
        compiler_params=pltpu.CompilerParams(
            dimension_semantics=("parallel", "arbitrary"),
        ),
        interpret=interpret,
    )(x_t, slots, ptr2, wv_t, b_v, W_det, bdet, pos_emb, wcat, bias)
    return out


def kernel(x_t, slots, ptr, W_v, b_v, W_det, b_det, pos_emb, W_ih, W_hh, b_ih, b_hh):
    ptr2 = ptr.astype(jnp.int32).reshape(B, 1)
    wv_t = W_v.T
    colscale = jnp.concatenate(
        [jnp.full((H,), 0.5, jnp.float32),
         jnp.full((H,), 0.5, jnp.float32),
         jnp.ones((H,), jnp.float32),
         jnp.full((H,), 0.5, jnp.float32)]
    )
    wcat = jnp.concatenate(
        [(W_ih.T * colscale[None, :]).astype(jnp.bfloat16),
         (W_hh.T * colscale[None, :]).astype(jnp.bfloat16)], axis=0)
    bias = ((b_ih + b_hh) * colscale).reshape(1, 4 * H)
    bv2 = b_v.reshape(1, D)
    bdet2 = b_det.reshape(1, 1)
    h_mem, new_slots, np2 = _run(x_t, slots, ptr2, wv_t, bv2, W_det, bdet2, pos_emb, wcat, bias)
    new_ptr = np2.reshape(B).astype(ptr.dtype)
    return (h_mem, new_slots, new_ptr)


# speedup vs baseline: 1.1509x; 1.1509x over previous
import functools

import jax
import jax.numpy as jnp
from jax.experimental import pallas as pl
from jax.experimental.pallas import tpu as pltpu

B = 512
D = 256
H = 512
S = 64
UNROLL = 16
NBLK = S // UNROLL
BSPLIT = 2
BB = B // BSPLIT


def _cell_kernel(
    x_ref,
    slots_ref,
    ptr_ref,
    wv_ref,
    bv_ref,
    wdet_ref,
    bdet_ref,
    pos_ref,
    wcat_ref,
    bias_ref,
    h_out_ref,
    ns_out_ref,
    np_out_ref,
    v_ref,
    m_ref,
    xh_ref,
    c_ref,
    pg_ref,
    xt_ref,
):
    t = pl.program_id(1)

    @pl.when(t == 0)
    def _prologue():
        x = x_ref[...]
        logit = jnp.sum(x * wdet_ref[...], axis=1, keepdims=True) + bdet_ref[...]
        ev = (jax.nn.sigmoid(logit) > 0.85)
        m_ref[...] = ev.astype(jnp.float32)
        v_ref[...] = jnp.dot(x, wv_ref[...], preferred_element_type=jnp.float32) + bv_ref[...]
        ptr = ptr_ref[...]
        np_out_ref[...] = jax.lax.rem(ptr + ev.astype(jnp.int32), jnp.int32(S))
        xh_ref[...] = jnp.zeros_like(xh_ref)
        c_ref[...] = jnp.zeros_like(c_ref)
        pg_ref[...] = bias_ref[...] + jnp.dot(
            pos_ref[...].astype(jnp.bfloat16), wcat_ref[0:D, :],
            preferred_element_type=jnp.float32)

    kidx = jax.lax.broadcasted_iota(jnp.int32, (BB, UNROLL, 1), 1) + t * UNROLL
    cond = jnp.logical_and(ptr_ref[...][:, :, None] == kidx,
                           m_ref[...][:, :, None] > 0.5)
    ns_block = jnp.where(cond, v_ref[...][:, None, :], slots_ref[...])
    ns_out_ref[...] = ns_block
    xt_ref[...] = jnp.swapaxes(ns_block.astype(jnp.bfloat16), 0, 1)

    for k in range(UNROLL):
        s = t * UNROLL + k
        xh_ref[:, 0:D] = xt_ref[k]
        gates = (
            jnp.dot(xh_ref[...], wcat_ref[...], preferred_element_type=jnp.float32)
            + pg_ref[pl.ds(s, 1), :]
        )
        tg = jnp.tanh(gates)
        i = 0.5 * tg[:, 0 * H:1 * H] + 0.5
        f = 0.5 * tg[:, 1 * H:2 * H] + 0.5
        g = tg[:, 2 * H:3 * H]
        o = 0.5 * tg[:, 3 * H:4 * H] + 0.5
        c = f * c_ref[...] + i * g
        c_ref[...] = c
        h_new = o * jnp.tanh(c)
        xh_ref[:, D:D + H] = h_new.astype(jnp.bfloat16)
        if k == UNROLL - 1:
            @pl.when(t == NBLK - 1)
            def _epilogue():
                h_out_ref[...] = h_new


@functools.partial(jax.jit, static_argnames=("interpret",))
def _run(x_t, slots, ptr2, wv_t, b_v, W_det, bdet, pos_emb, wcat, bias, interpret=False):
    shared = lambda shape: pl.BlockSpec(shape, lambda i, t: (0,) * len(shape))
    bhalf = lambda shape: pl.BlockSpec(shape, lambda i, t: (i,) + (0,) * (len(shape) - 1))
    out = pl.pallas_call(
        _cell_kernel,
        grid=(BSPLIT, NBLK),
        in_specs=[
            bhalf((BB, D)),
            pl.BlockSpec((BB, UNROLL, D), lambda i, t: (i, t, 0)),
            bhalf((BB, 1)),
            shared((D, D)),
            shared((1, D)),
            shared((1, D)),
            shared((1, 1)),
            shared((S, D)),
            shared((D + H, 4 * H)),
            shared((1, 4 * H)),
        ],
        out_specs=[
            bhalf((BB, H)),
            pl.BlockSpec((BB, UNROLL, D), lambda i, t: (i, t, 0)),
            bhalf((BB, 1)),
        ],
        out_shape=[
            jax.ShapeDtypeStruct((B, H), jnp.float32),
            jax.ShapeDtypeStruct((B, S, D), jnp.float32),
            jax.ShapeDtypeStruct((B, 1), jnp.int32),
        ],
        scratch_shapes=[
            pltpu.VMEM((BB, D), jnp.float32),
            pltpu.VMEM((BB, 1), jnp.float32),
            pltpu.VMEM((BB, D + H), jnp.bfloat16),
            pltpu.VMEM((BB, H), jnp.float32),
            pltpu.VMEM((S, 4 * H), jnp.float32),
            pltpu.VMEM((UNROLL, BB, D), jnp.bfloat16),
        ],
        compiler_params=pltpu.CompilerParams(
            dimension_semantics=("parallel", "arbitrary"),
        ),
        interpret=interpret,
    )(x_t, slots, ptr2, wv_t, b_v, W_det, bdet, pos_emb, wcat, bias)
    return out


def kernel(x_t, slots, ptr, W_v, b_v, W_det, b_det, pos_emb, W_ih, W_hh, b_ih, b_hh):
    ptr2 = ptr.astype(jnp.int32).reshape(B, 1)
    wv_t = W_v.T
    colscale = jnp.concatenate(
        [jnp.full((H,), 0.5, jnp.float32),
         jnp.full((H,), 0.5, jnp.float32),
         jnp.ones((H,), jnp.float32),
         jnp.full((H,), 0.5, jnp.float32)]
    )
    wcat = jnp.concatenate(
        [(W_ih.T * colscale[None, :]).astype(jnp.bfloat16),
         (W_hh.T * colscale[None, :]).astype(jnp.bfloat16)], axis=0)
    bias = ((b_ih + b_hh) * colscale).reshape(1, 4 * H)
    bv2 = b_v.reshape(1, D)
    bdet2 = b_det.reshape(1, 1)
    h_mem, new_slots, np2 = _run(x_t, slots, ptr2, wv_t, bv2, W_det, bdet2, pos_emb, wcat, bias)
    new_ptr = np2.reshape(B).astype(ptr.dtype)
    return (h_mem, new_slots, new_ptr)


# speedup vs baseline: 1.1563x; 1.0046x over previous
import functools

import jax
import jax.numpy as jnp
from jax.experimental import pallas as pl
from jax.experimental.pallas import tpu as pltpu

B = 512
D = 256
H = 512
S = 64
UNROLL = 16
NBLK = S // UNROLL
BSPLIT = 2
BB = B // BSPLIT


def _cell_kernel(
    x_ref,
    slots_ref,
    ptr_ref,
    wv_ref,
    bv_ref,
    wdet_ref,
    bdet_ref,
    pos_ref,
    wcat_ref,
    bias_ref,
    h_out_ref,
    ns_out_ref,
    np_out_ref,
    v_ref,
    m_ref,
    xh_ref,
    c_ref,
    pg_ref,
    xt_ref,
):
    t = pl.program_id(1)

    @pl.when(t == 0)
    def _prologue():
        x = x_ref[...]
        logit = jnp.sum(x * wdet_ref[...], axis=1, keepdims=True) + bdet_ref[...]
        ev = (jax.nn.sigmoid(logit) > 0.85)
        m_ref[...] = ev.astype(jnp.float32)
        v_ref[...] = jnp.dot(x, wv_ref[...], preferred_element_type=jnp.float32) + bv_ref[...]
        ptr = ptr_ref[...]
        np_out_ref[...] = jax.lax.rem(ptr + ev.astype(jnp.int32), jnp.int32(S))
        xh_ref[...] = jnp.zeros_like(xh_ref)
        c_ref[...] = jnp.zeros_like(c_ref)
        pg_ref[...] = bias_ref[...] + jnp.dot(
            pos_ref[...].astype(jnp.bfloat16), wcat_ref[0:D, :],
            preferred_element_type=jnp.float32)

    kidx = jax.lax.broadcasted_iota(jnp.int32, (BB, UNROLL, 1), 1) + t * UNROLL
    cond = jnp.logical_and(ptr_ref[...][:, :, None] == kidx,
                           m_ref[...][:, :, None] > 0.5)
    ns_block = jnp.where(cond, v_ref[...][:, None, :], slots_ref[...])
    ns_out_ref[...] = ns_block
    xt_ref[...] = jnp.swapaxes(ns_block.astype(jnp.bfloat16), 0, 1)

    for k in range(UNROLL):
        s = t * UNROLL + k
        xh_ref[:, 0:D] = xt_ref[k]
        gates = (
            jnp.dot(xh_ref[...], wcat_ref[...], preferred_element_type=jnp.float32)
            + pg_ref[pl.ds(s, 1), :]
        )
        tg = jnp.tanh(gates)
        ti = tg[:, 0 * H:1 * H] + 1.0
        tf = tg[:, 1 * H:2 * H] + 1.0
        g = tg[:, 2 * H:3 * H]
        to = tg[:, 3 * H:4 * H] + 1.0
        c = 0.5 * (tf * c_ref[...] + ti * g)
        c_ref[...] = c
        h_new = 0.5 * to * jnp.tanh(c)
        xh_ref[:, D:D + H] = h_new.astype(jnp.bfloat16)
        if k == UNROLL - 1:
            @pl.when(t == NBLK - 1)
            def _epilogue():
                h_out_ref[...] = h_new


@functools.partial(jax.jit, static_argnames=("interpret",))
def _run(x_t, slots, ptr2, wv_t, b_v, W_det, bdet, pos_emb, wcat, bias, interpret=False):
    shared = lambda shape: pl.BlockSpec(shape, lambda i, t: (0,) * len(shape))
    bhalf = lambda shape: pl.BlockSpec(shape, lambda i, t: (i,) + (0,) * (len(shape) - 1))
    out = pl.pallas_call(
        _cell_kernel,
        grid=(BSPLIT, NBLK),
        in_specs=[
            bhalf((BB, D)),
            pl.BlockSpec((BB, UNROLL, D), lambda i, t: (i, t, 0)),
            bhalf((BB, 1)),
            shared((D, D)),
            shared((1, D)),
            shared((1, D)),
            shared((1, 1)),
            shared((S, D)),
            shared((D + H, 4 * H)),
            shared((1, 4 * H)),
        ],
        out_specs=[
            bhalf((BB, H)),
            pl.BlockSpec((BB, UNROLL, D), lambda i, t: (i, t, 0)),
            bhalf((BB, 1)),
        ],
        out_shape=[
            jax.ShapeDtypeStruct((B, H), jnp.float32),
            jax.ShapeDtypeStruct((B, S, D), jnp.float32),
            jax.ShapeDtypeStruct((B, 1), jnp.int32),
        ],
        scratch_shapes=[
            pltpu.VMEM((BB, D), jnp.float32),
            pltpu.VMEM((BB, 1), jnp.float32),
            pltpu.VMEM((BB, D + H), jnp.bfloat16),
            pltpu.VMEM((BB, H), jnp.float32),
            pltpu.VMEM((S, 4 * H), jnp.float32),
            pltpu.VMEM((UNROLL, BB, D), jnp.bfloat16),
        ],
        compiler_params=pltpu.CompilerParams(
            dimension_semantics=("parallel", "arbitrary"),
        ),
        interpret=interpret,
    )(x_t, slots, ptr2, wv_t, b_v, W_det, bdet, pos_emb, wcat, bias)
    return out


def kernel(x_t, slots, ptr, W_v, b_v, W_det, b_det, pos_emb, W_ih, W_hh, b_ih, b_hh):
    ptr2 = ptr.astype(jnp.int32).reshape(B, 1)
    wv_t = W_v.T
    colscale = jnp.concatenate(
        [jnp.full((H,), 0.5, jnp.float32),
         jnp.full((H,), 0.5, jnp.float32),
         jnp.ones((H,), jnp.float32),
         jnp.full((H,), 0.5, jnp.float32)]
    )
    wcat = jnp.concatenate(
        [(W_ih.T * colscale[None, :]).astype(jnp.bfloat16),
         (W_hh.T * colscale[None, :]).astype(jnp.bfloat16)], axis=0)
    bias = ((b_ih + b_hh) * colscale).reshape(1, 4 * H)
    bv2 = b_v.reshape(1, D)
    bdet2 = b_det.reshape(1, 1)
    h_mem, new_slots, np2 = _run(x_t, slots, ptr2, wv_t, bv2, W_det, bdet2, pos_emb, wcat, bias)
    new_ptr = np2.reshape(B).astype(ptr.dtype)
    return (h_mem, new_slots, new_ptr)


# speedup vs baseline: 1.1599x; 1.0031x over previous
import functools

import jax
import jax.numpy as jnp
from jax.experimental import pallas as pl
from jax.experimental.pallas import tpu as pltpu

B = 512
D = 256
H = 512
S = 64
UNROLL = 16
NBLK = S // UNROLL
BSPLIT = 2
BB = B // BSPLIT


def _cell_kernel(
    x_ref,
    slots_ref,
    ptr_ref,
    wv_ref,
    bv_ref,
    wdet_ref,
    bdet_ref,
    pos_ref,
    wcat_ref,
    bias_ref,
    h_out_ref,
    ns_out_ref,
    np_out_ref,
    v_ref,
    m_ref,
    xh_ref,
    c_ref,
    pg_ref,
    xt_ref,
):
    t = pl.program_id(1)

    @pl.when(t == 0)
    def _prologue():
        x = x_ref[...]
        logit = jnp.sum(x * wdet_ref[...], axis=1, keepdims=True) + bdet_ref[...]
        ev = (jax.nn.sigmoid(logit) > 0.85)
        m_ref[...] = ev.astype(jnp.float32)
        v_ref[...] = jnp.dot(x, wv_ref[...], preferred_element_type=jnp.float32) + bv_ref[...]
        ptr = ptr_ref[...]
        np_out_ref[...] = jax.lax.rem(ptr + ev.astype(jnp.int32), jnp.int32(S))
        xh_ref[...] = jnp.zeros_like(xh_ref)
        c_ref[...] = jnp.zeros_like(c_ref)
        pg_ref[...] = bias_ref[...] + jnp.dot(
            pos_ref[...].astype(jnp.bfloat16), wcat_ref[0:D, :],
            preferred_element_type=jnp.float32)

    kidx = jax.lax.broadcasted_iota(jnp.int32, (BB, UNROLL, 1), 1) + t * UNROLL
    cond = jnp.logical_and(ptr_ref[...][:, :, None] == kidx,
                           m_ref[...][:, :, None] > 0.5)
    ns_block = jnp.where(cond, v_ref[...][:, None, :], slots_ref[...])
    ns_out_ref[...] = ns_block
    xt_ref[...] = jnp.swapaxes(ns_block.astype(jnp.bfloat16), 0, 1)

    for k in range(UNROLL):
        s = t * UNROLL + k
        xh_ref[:, 0:D] = xt_ref[k]
        gates = (
            jnp.dot(xh_ref[...], wcat_ref[...], preferred_element_type=jnp.float32)
            + pg_ref[pl.ds(s, 1), :]
        )
        tg = jnp.tanh(gates)
        ti = tg[:, 0 * H:1 * H] + 1.0
        tf = tg[:, 1 * H:2 * H] + 1.0
        g = tg[:, 2 * H:3 * H]
        to = tg[:, 3 * H:4 * H] + 1.0
        c = 0.5 * (tf * c_ref[...] + ti * g)
        c_ref[...] = c
        h_new = 0.5 * to * jnp.tanh(c)
        xh_ref[:, D:D + H] = h_new.astype(jnp.bfloat16)
        if k == UNROLL - 1:
            @pl.when(t == NBLK - 1)
            def _epilogue():
                h_out_ref[...] = h_new


@jax.jit
def _run(x_t, slots, ptr2, wv_t, b_v, W_det, bdet, pos_emb, wcat, bias):
    shared = lambda shape: pl.BlockSpec(shape, lambda i, t: (0,) * len(shape))
    bhalf = lambda shape: pl.BlockSpec(shape, lambda i, t: (i,) + (0,) * (len(shape) - 1))
    out = pl.pallas_call(
        _cell_kernel,
        grid=(BSPLIT, NBLK),
        in_specs=[
            bhalf((BB, D)),
            pl.BlockSpec((BB, UNROLL, D), lambda i, t: (i, t, 0)),
            bhalf((BB, 1)),
            shared((D, D)),
            shared((1, D)),
            shared((1, D)),
            shared((1, 1)),
            shared((S, D)),
            shared((D + H, 4 * H)),
            shared((1, 4 * H)),
        ],
        out_specs=[
            bhalf((BB, H)),
            pl.BlockSpec((BB, UNROLL, D), lambda i, t: (i, t, 0)),
            bhalf((BB, 1)),
        ],
        out_shape=[
            jax.ShapeDtypeStruct((B, H), jnp.float32),
            jax.ShapeDtypeStruct((B, S, D), jnp.float32),
            jax.ShapeDtypeStruct((B, 1), jnp.int32),
        ],
        scratch_shapes=[
            pltpu.VMEM((BB, D), jnp.float32),
            pltpu.VMEM((BB, 1), jnp.float32),
            pltpu.VMEM((BB, D + H), jnp.bfloat16),
            pltpu.VMEM((BB, H), jnp.float32),
            pltpu.VMEM((S, 4 * H), jnp.float32),
            pltpu.VMEM((UNROLL, BB, D), jnp.bfloat16),
        ],
        compiler_params=pltpu.CompilerParams(
            dimension_semantics=("parallel", "arbitrary"),
        ),
    )(x_t, slots, ptr2, wv_t, b_v, W_det, bdet, pos_emb, wcat, bias)
    return out


def kernel(x_t, slots, ptr, W_v, b_v, W_det, b_det, pos_emb, W_ih, W_hh, b_ih, b_hh):
    ptr2 = ptr.astype(jnp.int32).reshape(B, 1)
    wv_t = W_v.T
    colscale = jnp.concatenate(
        [jnp.full((H,), 0.5, jnp.float32),
         jnp.full((H,), 0.5, jnp.float32),
         jnp.ones((H,), jnp.float32),
         jnp.full((H,), 0.5, jnp.float32)]
    )
    wcat = jnp.concatenate(
        [(W_ih.T * colscale[None, :]).astype(jnp.bfloat16),
         (W_hh.T * colscale[None, :]).astype(jnp.bfloat16)], axis=0)
    bias = ((b_ih + b_hh) * colscale).reshape(1, 4 * H)
    bv2 = b_v.reshape(1, D)
    bdet2 = b_det.reshape(1, 1)
    h_mem, new_slots, np2 = _run(x_t, slots, ptr2, wv_t, bv2, W_det, bdet2, pos_emb, wcat, bias)
    new_ptr = np2.reshape(B).astype(ptr.dtype)
    return (h_mem, new_slots, new_ptr)
